# Initial kernel scaffold; baseline (speedup 1.0000x reference)
import jax
import jax.numpy as jnp
from jax.experimental import pallas as pl


def kernel(token_ids, row_table, col_table):
    raise NotImplementedError("write your pallas kernel here")


# speedup vs baseline: 6.1481x; 6.1481x over previous
import functools

import jax
import jax.numpy as jnp
from jax import lax
from jax.experimental import pallas as pl
from jax.experimental.pallas import tpu as pltpu
from jax.experimental.pallas import tpu_sc as plsc

NC = 2
NS = 16
LANES = 16
NW = NC * NS

TABLE = 1000
D = 64
N_TOKENS = 4096 * 200
PER_W = N_TOKENS // NW
CHUNK = 512
G = 128
NG = CHUNK // G
N_CHUNKS = PER_W // CHUNK


def _split_ids(v):
    r0 = (v.astype(jnp.float32) * (1.0 / TABLE)).astype(jnp.int32)
    d0 = v - r0 * TABLE
    r = jnp.where(d0 >= TABLE, r0 + 1, jnp.where(d0 < 0, r0 - 1, r0))
    c = v - r * TABLE
    return r, c


_mesh = plsc.VectorSubcoreMesh(
    core_axis_name="c", subcore_axis_name="s", num_cores=NC, num_subcores=NS
)


@functools.partial(
    pl.kernel,
    out_type=jax.ShapeDtypeStruct((N_TOKENS, D), jnp.float32),
    mesh=_mesh,
    compiler_params=pltpu.CompilerParams(use_tc_tiling_on_sc=False),
    scratch_types=[
        pltpu.VMEM((CHUNK,), jnp.int32),
        pltpu.VMEM((NG, G), jnp.int32),
        pltpu.VMEM((NG, G), jnp.int32),
        pltpu.VMEM((CHUNK, D), jnp.float32),
        pltpu.VMEM((CHUNK, D), jnp.float32),
        pltpu.SemaphoreType.DMA,
    ],
)
def _sc_lookup(ids_hbm, row_hbm, col_hbm, out_hbm, ids_v, ridx, cidx, rows,
               cols, sem):
    wid = lax.axis_index("s") * NC + lax.axis_index("c")
    base_w = wid * PER_W

    def chunk_body(ch, carry):
        base = base_w + ch * CHUNK
        pltpu.sync_copy(ids_hbm.at[pl.ds(base, CHUNK)], ids_v)

        for j in range(NG):
            def grp(kk, c2, j=j):
                v = ids_v[pl.ds(j * G + kk * LANES, LANES)]
                r, c = _split_ids(v)
                ridx[j, pl.ds(kk * LANES, LANES)] = r
                cidx[j, pl.ds(kk * LANES, LANES)] = c
                return c2
            lax.fori_loop(0, G // LANES, grp, 0)

        copies = []
        for j in range(NG):
            copies.append(
                pltpu.async_copy(row_hbm.at[ridx.at[j]],
                                 rows.at[pl.ds(j * G, G)], sem))
            copies.append(
                pltpu.async_copy(col_hbm.at[cidx.at[j]],
                                 cols.at[pl.ds(j * G, G)], sem))
        for cp in copies:
            cp.wait()

        def add_body(i, c2):
            for dd in range(D // LANES):
                rows[i, pl.ds(dd * LANES, LANES)] = (
                    rows[i, pl.ds(dd * LANES, LANES)]
                    + cols[i, pl.ds(dd * LANES, LANES)])
            return c2
        lax.fori_loop(0, CHUNK, add_body, 0)

        pltpu.sync_copy(rows, out_hbm.at[pl.ds(base, CHUNK)])
        return carry

    lax.fori_loop(0, N_CHUNKS, chunk_body, 0)


def kernel(token_ids, row_table, col_table):
    b, h = token_ids.shape
    ids = token_ids.reshape(b * h).astype(jnp.int32)
    out = _sc_lookup(ids, row_table, col_table)
    return out.reshape(b, h, D)
